# baseline (speedup 1.0000x reference)
import functools

import jax
import jax.numpy as jnp
import numpy as np
from jax import lax
from jax.experimental import pallas as pl
from jax.experimental.pallas import tpu as pltpu
from jax.experimental.pallas import tpu_sc as plsc

_T = 1000
_NUM_NODES = 4096
_NPE = _NUM_NODES * (_NUM_NODES - 1) // 2
_EM0, _EM1 = 0.9, 0.1


def _cos_schedule(T, s=0.008):
    num_steps = T + 2
    t = np.linspace(0, num_steps, num_steps)
    alpha_bars = np.cos(0.5 * np.pi * (t / num_steps + s) / (1 + s)) ** 2
    alpha_bars = alpha_bars / alpha_bars[0]
    alphas = alpha_bars[1:] / alpha_bars[:-1]
    betas = np.clip(1 - alphas, 0, 0.9999)
    alphas = 1 - betas
    log_alphas = np.log(alphas)
    log_alpha_bars = np.cumsum(log_alphas)
    alpha_bars = np.exp(log_alpha_bars)
    return alpha_bars.astype(np.float32), alphas.astype(np.float32)


_AB_NP, _AL_NP = _cos_schedule(_T)

_NB = _NPE // 128
_NW = 32
_NB_HI = 2048
_NB_LO = 2047
_CB = 64
_NCH = 32
_CE = _CB * 128
_CF = _CB * 256


def _posterior_sc(consts, e_t, logits_phys):
    mesh = plsc.VectorSubcoreMesh(core_axis_name="c", subcore_axis_name="s")

    @functools.partial(
        pl.kernel,
        mesh=mesh,
        out_type=jax.ShapeDtypeStruct((2 * _NPE,), jnp.float32),
        compiler_params=pltpu.CompilerParams(needs_layout_passes=False),
        scratch_types=[
            pltpu.VMEM((8, 16), jnp.float32),
            pltpu.VMEM((_CE,), jnp.int32),
            pltpu.VMEM((_CE,), jnp.int32),
            pltpu.VMEM((_CF,), jnp.float32),
            pltpu.VMEM((_CF,), jnp.float32),
            pltpu.VMEM((_CF,), jnp.float32),
            pltpu.VMEM((_CF,), jnp.float32),
            pltpu.SemaphoreType.DMA,
            pltpu.SemaphoreType.DMA,
            pltpu.SemaphoreType.DMA,
            pltpu.SemaphoreType.DMA,
        ],
    )
    def k(consts_hbm, e_hbm, l_hbm, out_hbm, cbuf, eb0, eb1, lb0, lb1,
          ob0, ob1, si0, si1, so0, so1):
        wid = lax.axis_index("s") * 2 + lax.axis_index("c")
        base_b = wid * _NB_HI - jnp.maximum(wid - 16, 0)
        nb = _NB_HI - (wid >= 16).astype(jnp.int32)
        pltpu.sync_copy(consts_hbm, cbuf)

        b0 = cbuf[0, :]
        c0 = cbuf[1, :]
        b1 = cbuf[2, :]
        c1 = cbuf[3, :]
        a0 = cbuf[4, :]
        da0 = cbuf[5, :]
        a1 = cbuf[6, :]
        da1 = cbuf[7, :]

        banks = ((eb0, lb0, ob0, si0, so0), (eb1, lb1, ob1, si1, so1))

        def chunk_off(c):
            return base_b + jnp.minimum(c * _CB, nb - _CB)

        def start_in(c, bank):
            eb, lb, _, si, _ = banks[bank]
            ob_ = chunk_off(c)
            pltpu.async_copy(e_hbm.at[pl.ds(ob_ * 128, _CE)], eb, si)
            pltpu.async_copy(l_hbm.at[pl.ds(ob_ * 256, _CF)], lb, si)

        def wait_in(bank):
            eb, lb, _, si, _ = banks[bank]
            pltpu.make_async_copy(e_hbm.at[pl.ds(0, _CE)], eb, si).wait()
            pltpu.make_async_copy(l_hbm.at[pl.ds(0, _CF)], lb, si).wait()

        def start_out(c, bank):
            _, _, ob, _, so = banks[bank]
            ob_ = chunk_off(c)
            pltpu.async_copy(ob, out_hbm.at[pl.ds(ob_ * 256, _CF)], so)

        def wait_out(bank):
            _, _, ob, _, so = banks[bank]
            pltpu.make_async_copy(ob, out_hbm.at[pl.ds(0, _CF)], so).wait()

        def compute(bank):
            eb, lb, ob, _, _ = banks[bank]

            @plsc.parallel_loop(0, _CB, 1, unroll=4)
            def _(ib):
                for j in range(8):
                    off = ib * 256 + j * 16
                    eoff = ib * 128 + j * 16
                    l0 = lb[pl.ds(off, 16)]
                    l1 = lb[pl.ds(off + 128, 16)]
                    e = eb[pl.ds(eoff, 16)]
                    ef = e.astype(jnp.float32)
                    u = jnp.exp(l1 - l0)
                    m0 = b0 + u * c0
                    m1 = b1 + u * c1
                    q0 = a0 + ef * da0
                    q1 = a1 + ef * da1
                    v0 = q0 * m0
                    v1 = q1 * m1
                    r = 1.0 / (v0 + v1)
                    ob[pl.ds(off, 16)] = v0 * r
                    ob[pl.ds(off + 128, 16)] = v1 * r

        start_in(0, 0)

        def pair_body(g, carry):
            for b in range(2):
                c = g * 2 + b

                @pl.when(c + 1 < _NCH)
                def _():
                    start_in(c + 1, 1 - b)

                wait_in(b)

                @pl.when(g >= 1)
                def _():
                    wait_out(b)

                compute(b)
                start_out(c, b)
            return carry

        lax.fori_loop(0, _NCH // 2, pair_body, 0)
        wait_out(0)
        wait_out(1)

    return k(consts, e_t, logits_phys)


def kernel(E_t, pred_E_logits, t):
    t = jnp.asarray(t)
    s = jnp.maximum(t - 1, 0)
    ab_s = jnp.asarray(_AB_NP)[s]
    a_t = jnp.asarray(_AL_NP)[t]
    is0 = t == 0
    one = jnp.float32(1.0)
    ab = jnp.where(is0, one, ab_s)
    qt00 = jnp.where(is0, one, a_t + (1 - a_t) * _EM0)
    qt01 = jnp.where(is0, one, (1 - a_t) * _EM1)
    qt10 = jnp.where(is0, one, (1 - a_t) * _EM0)
    qt11 = jnp.where(is0, one, a_t + (1 - a_t) * _EM1)
    qb00 = ab + (1 - ab) * _EM0
    qb01 = (1 - ab) * _EM1
    qb10 = (1 - ab) * _EM0
    qb11 = ab + (1 - ab) * _EM1
    consts = jnp.stack(
        [qb00, qb10, qb01, qb11, qt00, qt01 - qt00, qt10, qt11 - qt10]
    ).astype(jnp.float32)
    consts = jnp.broadcast_to(consts[:, None], (8, 16))
    logits_phys = (
        pred_E_logits.reshape(_NB, 128, 2).transpose(0, 2, 1).reshape(-1)
    )
    out_flat = _posterior_sc(consts, E_t, logits_phys)
    return out_flat.reshape(_NB, 2, 128).transpose(0, 2, 1).reshape(_NPE, 2)


# speedup vs baseline: 1.0711x; 1.0711x over previous
import functools

import jax
import jax.numpy as jnp
import numpy as np
from jax import lax
from jax.experimental import pallas as pl
from jax.experimental.pallas import tpu as pltpu
from jax.experimental.pallas import tpu_sc as plsc

_T = 1000
_NUM_NODES = 4096
_NPE = _NUM_NODES * (_NUM_NODES - 1) // 2
_EM0, _EM1 = 0.9, 0.1


def _cos_schedule(T, s=0.008):
    num_steps = T + 2
    t = np.linspace(0, num_steps, num_steps)
    alpha_bars = np.cos(0.5 * np.pi * (t / num_steps + s) / (1 + s)) ** 2
    alpha_bars = alpha_bars / alpha_bars[0]
    alphas = alpha_bars[1:] / alpha_bars[:-1]
    betas = np.clip(1 - alphas, 0, 0.9999)
    alphas = 1 - betas
    log_alphas = np.log(alphas)
    log_alpha_bars = np.cumsum(log_alphas)
    alpha_bars = np.exp(log_alpha_bars)
    return alpha_bars.astype(np.float32), alphas.astype(np.float32)


_AB_NP, _AL_NP = _cos_schedule(_T)

_NB = _NPE // 128
_NW = 32
_NB_HI = 2048
_NB_LO = 2047
_CB = 64
_NCH = 32
_CE = _CB * 128
_CF = _CB * 256


def _posterior_sc(consts, e_t, logits_phys):
    mesh = plsc.VectorSubcoreMesh(core_axis_name="c", subcore_axis_name="s")

    @functools.partial(
        pl.kernel,
        mesh=mesh,
        out_type=jax.ShapeDtypeStruct((2 * _NPE,), jnp.float32),
        compiler_params=pltpu.CompilerParams(needs_layout_passes=False),
        scratch_types=[
            pltpu.VMEM((8, 16), jnp.float32),
            pltpu.VMEM((_CE,), jnp.int32),
            pltpu.VMEM((_CE,), jnp.int32),
            pltpu.VMEM((_CF,), jnp.float32),
            pltpu.VMEM((_CF,), jnp.float32),
            pltpu.VMEM((_CF,), jnp.float32),
            pltpu.VMEM((_CF,), jnp.float32),
            pltpu.SemaphoreType.DMA,
            pltpu.SemaphoreType.DMA,
            pltpu.SemaphoreType.DMA,
            pltpu.SemaphoreType.DMA,
        ],
    )
    def k(consts_hbm, e_hbm, l_hbm, out_hbm, cbuf, eb0, eb1, lb0, lb1,
          ob0, ob1, si0, si1, so0, so1):
        wid = lax.axis_index("s") * 2 + lax.axis_index("c")
        base_b = wid * _NB_HI - jnp.maximum(wid - 16, 0)
        nb = _NB_HI - (wid >= 16).astype(jnp.int32)
        pltpu.sync_copy(consts_hbm, cbuf)

        b0 = cbuf[0, :]
        c0 = cbuf[1, :]
        b1 = cbuf[2, :]
        c1 = cbuf[3, :]
        a0 = cbuf[4, :]
        da0 = cbuf[5, :]
        a1 = cbuf[6, :]
        da1 = cbuf[7, :]

        banks = ((eb0, lb0, ob0, si0, so0), (eb1, lb1, ob1, si1, so1))

        def chunk_off(c):
            return base_b + jnp.minimum(c * _CB, nb - _CB)

        def start_in(c, bank):
            eb, lb, _, si, _ = banks[bank]
            ob_ = chunk_off(c)
            pltpu.async_copy(e_hbm.at[pl.ds(ob_ * 128, _CE)], eb, si)
            pltpu.async_copy(l_hbm.at[pl.ds(ob_ * 256, _CF)], lb, si)

        def wait_in(bank):
            eb, lb, _, si, _ = banks[bank]
            pltpu.make_async_copy(e_hbm.at[pl.ds(0, _CE)], eb, si).wait()
            pltpu.make_async_copy(l_hbm.at[pl.ds(0, _CF)], lb, si).wait()

        def start_out(c, bank):
            _, _, ob, _, so = banks[bank]
            ob_ = chunk_off(c)
            pltpu.async_copy(ob, out_hbm.at[pl.ds(ob_ * 256, _CF)], so)

        def wait_out(bank):
            _, _, ob, _, so = banks[bank]
            pltpu.make_async_copy(ob, out_hbm.at[pl.ds(0, _CF)], so).wait()

        def compute(bank):
            eb, lb, ob, _, _ = banks[bank]

            @plsc.parallel_loop(0, _CB, 1, unroll=4)
            def _(ib):
                for j in range(8):
                    off = ib * 256 + j * 16
                    eoff = ib * 128 + j * 16
                    l0 = lb[pl.ds(off, 16)]
                    l1 = lb[pl.ds(off + 128, 16)]
                    e = eb[pl.ds(eoff, 16)]
                    ef = e.astype(jnp.float32)
                    u = jnp.exp(jnp.minimum(l1 - l0, 60.0))
                    m0 = b0 + u * c0
                    m1 = b1 + u * c1
                    q0 = a0 + ef * da0
                    q1 = a1 + ef * da1
                    v0 = q0 * m0
                    v1 = q1 * m1
                    r = 1.0 / (v0 + v1 + 1e-10)
                    ob[pl.ds(off, 16)] = v0 * r
                    ob[pl.ds(off + 128, 16)] = v1 * r

        start_in(0, 0)

        def pair_body(g, carry):
            for b in range(2):
                c = g * 2 + b

                @pl.when(c + 1 < _NCH)
                def _():
                    start_in(c + 1, 1 - b)

                wait_in(b)

                @pl.when(g >= 1)
                def _():
                    wait_out(b)

                compute(b)
                start_out(c, b)
            return carry

        lax.fori_loop(0, _NCH // 2, pair_body, 0)
        wait_out(0)
        wait_out(1)

    return k(consts, e_t, logits_phys)


def kernel(E_t, pred_E_logits, t):
    t = jnp.asarray(t)
    s = jnp.maximum(t - 1, 0)
    ab_s = jnp.asarray(_AB_NP)[s]
    a_t = jnp.asarray(_AL_NP)[t]
    is0 = t == 0
    one = jnp.float32(1.0)
    ab = jnp.where(is0, one, ab_s)
    qt00 = jnp.where(is0, one, a_t + (1 - a_t) * _EM0)
    qt01 = jnp.where(is0, one, (1 - a_t) * _EM1)
    qt10 = jnp.where(is0, one, (1 - a_t) * _EM0)
    qt11 = jnp.where(is0, one, a_t + (1 - a_t) * _EM1)
    qb00 = ab + (1 - ab) * _EM0
    qb01 = (1 - ab) * _EM1
    qb10 = (1 - ab) * _EM0
    qb11 = ab + (1 - ab) * _EM1
    consts = jnp.stack(
        [qb00, qb10, qb01, qb11, qt00, qt01 - qt00, qt10, qt11 - qt10]
    ).astype(jnp.float32)
    consts = jnp.broadcast_to(consts[:, None], (8, 16))
    logits_phys = (
        pred_E_logits.reshape(_NB, 128, 2).transpose(0, 2, 1).reshape(-1)
    )
    out_flat = _posterior_sc(consts, E_t, logits_phys)
    return out_flat.reshape(_NB, 2, 128).transpose(0, 2, 1).reshape(_NPE, 2)
